# initial kernel scaffold (unmeasured)
import jax
import jax.numpy as jnp
from jax import lax
from jax.experimental import pallas as pl
from jax.experimental.pallas import tpu as pltpu

N_DEV = 16
W = 64


def kernel(x, A, B, C):
    b, s, d = x.shape
    n = A.shape[1]

    dA = jnp.exp(A).T
    Bn = jnp.transpose(B, (0, 2, 1))
    Cn = jnp.transpose(C, (0, 2, 1))

    def body(x_ref, dA_ref, Bn_ref, Cn_ref, out_ref,
             send_buf, recv_buf, send_sem, recv_sem):
        my = lax.axis_index("i")
        dAv = dA_ref[...][None]

        def update(t, h):
            x_t = x_ref[:, pl.ds(t, 1), :]
            b_t = Bn_ref[:, :, pl.ds(t, 1)]
            return h * dAv + x_t * b_t

        h_tail = lax.fori_loop(
            0, W, lambda i, h: update(s - W + i, h),
            jnp.zeros((b, n, d), jnp.float32))
        send_buf[...] = h_tail

        rdma = pltpu.make_async_remote_copy(
            src_ref=send_buf,
            dst_ref=recv_buf,
            send_sem=send_sem,
            recv_sem=recv_sem,
            device_id=(jnp.minimum(my + 1, N_DEV - 1),),
            device_id_type=pl.DeviceIdType.MESH,
        )

        @pl.when(my < N_DEV - 1)
        def _():
            rdma.start()

        @pl.when(my > 0)
        def _():
            rdma.wait_recv()

        h0 = jnp.where(my > 0, recv_buf[...],
                       jnp.zeros((b, n, d), jnp.float32))

        def main_step(t, h):
            h = update(t, h)
            c_t = Cn_ref[:, :, pl.ds(t, 1)]
            out_ref[:, pl.ds(t, 1), :] = jnp.sum(
                h * c_t, axis=1, keepdims=True)
            return h

        lax.fori_loop(0, s, main_step, h0)

        @pl.when(my < N_DEV - 1)
        def _():
            rdma.wait_send()

    return pl.pallas_call(
        body,
        out_shape=jax.ShapeDtypeStruct((b, s, d), jnp.float32),
        in_specs=[pl.BlockSpec(memory_space=pltpu.VMEM)] * 4,
        out_specs=pl.BlockSpec(memory_space=pltpu.VMEM),
        scratch_shapes=[
            pltpu.VMEM((b, n, d), jnp.float32),
            pltpu.VMEM((b, n, d), jnp.float32),
            pltpu.SemaphoreType.DMA,
            pltpu.SemaphoreType.DMA,
        ],
    )(x, dA, Bn, Cn)


# baseline (device time: 73330 ns/iter reference)
import jax
import jax.numpy as jnp
from jax import lax
from jax.experimental import pallas as pl
from jax.experimental.pallas import tpu as pltpu

N_DEV = 16
W = 64
TB = 8


def kernel(x, A, B, C):
    b, s, d = x.shape
    n = A.shape[1]

    dA = jnp.exp(A).T

    def body(x_ref, dA_ref, B_ref, C_ref, out_ref,
             send_buf, recv_buf, send_sem, recv_sem):
        my = lax.axis_index("i")
        dAv = dA_ref[...][None]

        def load_block(t0):
            t0 = pl.multiple_of(t0, TB)
            xb = x_ref[:, pl.ds(t0, TB), :]
            bb = jnp.swapaxes(B_ref[:, pl.ds(t0, TB), :], 1, 2)
            cb = jnp.swapaxes(C_ref[:, pl.ds(t0, TB), :], 1, 2)
            return xb, bb, cb

        def tail_block(i, h):
            xb, bb, _ = load_block(s - W + i * TB)
            for j in range(TB):
                h = h * dAv + xb[:, j:j + 1, :] * bb[:, :, j:j + 1]
            return h

        h_tail = lax.fori_loop(0, W // TB, tail_block,
                               jnp.zeros((b, n, d), jnp.float32))
        send_buf[...] = h_tail

        rdma = pltpu.make_async_remote_copy(
            src_ref=send_buf,
            dst_ref=recv_buf,
            send_sem=send_sem,
            recv_sem=recv_sem,
            device_id=(jnp.minimum(my + 1, N_DEV - 1),),
            device_id_type=pl.DeviceIdType.MESH,
        )

        @pl.when(my < N_DEV - 1)
        def _():
            rdma.start()

        @pl.when(my > 0)
        def _():
            rdma.wait_recv()

        h0 = jnp.where(my > 0, recv_buf[...],
                       jnp.zeros((b, n, d), jnp.float32))

        def main_block(i, h):
            t0 = i * TB
            xb, bb, cb = load_block(t0)
            ys = []
            for j in range(TB):
                h = h * dAv + xb[:, j:j + 1, :] * bb[:, :, j:j + 1]
                ys.append(jnp.sum(h * cb[:, :, j:j + 1], axis=1,
                                  keepdims=True))
            out_ref[:, pl.ds(pl.multiple_of(t0, TB), TB), :] = (
                jnp.concatenate(ys, axis=1))
            return h

        lax.fori_loop(0, s // TB, main_block, h0)

        @pl.when(my < N_DEV - 1)
        def _():
            rdma.wait_send()

    return pl.pallas_call(
        body,
        out_shape=jax.ShapeDtypeStruct((b, s, d), jnp.float32),
        in_specs=[pl.BlockSpec(memory_space=pltpu.VMEM)] * 4,
        out_specs=pl.BlockSpec(memory_space=pltpu.VMEM),
        scratch_shapes=[
            pltpu.VMEM((b, n, d), jnp.float32),
            pltpu.VMEM((b, n, d), jnp.float32),
            pltpu.SemaphoreType.DMA,
            pltpu.SemaphoreType.DMA,
        ],
    )(x, dA, B, C)
